# Initial kernel scaffold; baseline (speedup 1.0000x reference)
import jax
import jax.numpy as jnp
from jax.experimental import pallas as pl


def kernel(q, k, v):
    raise NotImplementedError("write your pallas kernel here")


# speedup vs baseline: 7.6856x; 7.6856x over previous
import math

import jax
import jax.numpy as jnp
from jax.experimental import pallas as pl

_SM_NEG = -1e30


def _attn_kernel(q_ref, k_ref, v_ref, o_ref):
    S = q_ref.shape[2]
    D = q_ref.shape[3]
    L = S.bit_length() - 1
    scale = 1.0 / math.sqrt(D)

    q = q_ref[0, 0, :, :]
    k = k_ref[0, 0, :, :]
    v = v_ref[0, 0, :, :]

    row = jax.lax.broadcasted_iota(jnp.int32, (S, 1), 0)

    m = jnp.sum(q * k, axis=-1, keepdims=True) * scale
    d = jnp.ones_like(m)
    acc = v

    kl, vl = k, v
    for l in range(L):
        n = S >> l
        kr = kl.reshape(n // 2, 2, D)
        vr = vl.reshape(n // 2, 2, D)
        kc0 = kr[:, 0, :]
        kc1 = kr[:, 1, :]
        vc0 = vr[:, 0, :]
        vc1 = vr[:, 1, :]

        rep = 1 << (l + 1)
        nbr_k = jnp.broadcast_to(kc0[:, None, :], (n // 2, rep, D)).reshape(S, D)
        nbr_v = jnp.broadcast_to(vc0[:, None, :], (n // 2, rep, D)).reshape(S, D)
        s = jnp.sum(q * nbr_k, axis=-1, keepdims=True) * scale
        allowed = ((row >> l) & 1) == 1
        s = jnp.where(allowed, s, _SM_NEG)
        m_new = jnp.maximum(m, s)
        alpha = jnp.exp(m - m_new)
        beta = jnp.exp(s - m_new)
        d = d * alpha + beta
        acc = acc * alpha + beta * nbr_v
        m = m_new

        if l + 1 < L:
            kp = 0.5 * (kc0 + kc1)
            vp = 0.5 * (vc0 + vc1)
            ss = jnp.sum(kp * kp, axis=-1, keepdims=True) * scale
            s0 = jnp.sum(kp * kc0, axis=-1, keepdims=True) * scale
            s1 = jnp.sum(kp * kc1, axis=-1, keepdims=True) * scale
            mm = jnp.maximum(jnp.maximum(ss, s0), s1)
            es = jnp.exp(ss - mm)
            e0 = jnp.exp(s0 - mm)
            e1 = jnp.exp(s1 - mm)
            den = es + e0 + e1
            vl = (es * vp + e0 * vc0 + e1 * vc1) / den
            kl = kp

    o_ref[0, 0, :, :] = acc / d


@jax.jit
def kernel(q, k, v):
    B, S, H, D = q.shape
    qt = q.transpose(0, 2, 1, 3)
    kt = k.transpose(0, 2, 1, 3)
    vt = v.transpose(0, 2, 1, 3)
    spec = pl.BlockSpec((1, 1, S, D), lambda b, h: (b, h, 0, 0))
    out = pl.pallas_call(
        _attn_kernel,
        grid=(B, H),
        in_specs=[spec, spec, spec],
        out_specs=spec,
        out_shape=jax.ShapeDtypeStruct((B, H, S, D), q.dtype),
    )(qt, kt, vt)
    return out.transpose(0, 2, 1, 3)


# speedup vs baseline: 9.6973x; 1.2617x over previous
import math

import jax
import jax.numpy as jnp
from jax.experimental import pallas as pl


def _attn_kernel(q_ref, k_ref, v_ref, o_ref):
    S = q_ref.shape[2]
    D = q_ref.shape[3]
    L = S.bit_length() - 1
    scale = 1.0 / math.sqrt(D)

    q = q_ref[0, 0, :, :]
    k = k_ref[0, 0, :, :]
    v = v_ref[0, 0, :, :]

    ones = jnp.ones((D, 1), jnp.float32)
    dnums = (((1,), (0,)), ((), ()))

    def rowdot(a, b):
        return jax.lax.dot_general(a * b, ones, dnums,
                                   preferred_element_type=jnp.float32)

    row = jax.lax.broadcasted_iota(jnp.int32, (S, 1), 0)

    m = rowdot(q, k) * scale
    d = jnp.ones_like(m)
    acc = v

    kl, vl = k, v
    for l in range(L):
        n = S >> l
        kr = kl.reshape(n // 2, 2, D)
        vr = vl.reshape(n // 2, 2, D)
        kc0 = kr[:, 0, :]
        kc1 = kr[:, 1, :]
        vc0 = vr[:, 0, :]
        vc1 = vr[:, 1, :]

        rep = 1 << (l + 1)
        nbr_k = jnp.broadcast_to(kc0[:, None, :], (n // 2, rep, D)).reshape(S, D)
        nbr_v = jnp.broadcast_to(vc0[:, None, :], (n // 2, rep, D)).reshape(S, D)
        s = rowdot(q, nbr_k) * scale
        allowed = ((row >> l) & 1) == 1
        e = jnp.where(allowed, jnp.exp(s - m), 0.0)
        d = d + e
        acc = acc + e * nbr_v

        if l + 1 < L:
            kp = 0.5 * (kc0 + kc1)
            s0 = rowdot(kp, kc0) * scale
            s1 = rowdot(kp, kc1) * scale
            ss = 0.5 * (s0 + s1)
            mm = jnp.maximum(jnp.maximum(ss, s0), s1)
            es = jnp.exp(ss - mm)
            e0 = jnp.exp(s0 - mm)
            e1 = jnp.exp(s1 - mm)
            rden = 1.0 / (es + e0 + e1)
            c0 = (0.5 * es + e0) * rden
            c1 = (0.5 * es + e1) * rden
            vl = c0 * vc0 + c1 * vc1
            kl = kp

    o_ref[0, 0, :, :] = acc * (1.0 / d)


@jax.jit
def kernel(q, k, v):
    B, S, H, D = q.shape
    qt = q.transpose(0, 2, 1, 3)
    kt = k.transpose(0, 2, 1, 3)
    vt = v.transpose(0, 2, 1, 3)
    spec = pl.BlockSpec((1, 1, S, D), lambda b, h: (b, h, 0, 0))
    out = pl.pallas_call(
        _attn_kernel,
        grid=(B, H),
        in_specs=[spec, spec, spec],
        out_specs=spec,
        out_shape=jax.ShapeDtypeStruct((B, H, S, D), q.dtype),
    )(qt, kt, vt)
    return out.transpose(0, 2, 1, 3)
